# baseline (speedup 1.0000x reference)
import functools

import jax
import jax.numpy as jnp
from jax import lax
from jax.experimental import pallas as pl
from jax.experimental.pallas import tpu as pltpu
from jax.experimental.pallas import tpu_sc as plsc

N_NODES = 10000
N_EDGES = 320000
D = 128
C = 128
INV_SQRT_C = 1.0 / (C ** 0.5)

NC = 2
NS = 16
NW = NC * NS
E_PER_W = N_EDGES // NW
E_SUB = 40
N_SUB = E_PER_W // E_SUB
N_PAIR = N_SUB // 2
N_PAD = 10240
ROWS_PER_TILE = N_PAD // NS
ZROWS = 128

BLK = 400


def _proj_body(x_ref, wq_ref, bq_ref, wk_ref, bk_ref, wv_ref, bv_ref,
               q_ref, kv_ref):
    xb = x_ref[...]
    q = jnp.dot(xb, wq_ref[...], preferred_element_type=jnp.float32) + bq_ref[...]
    q_ref[...] = q * INV_SQRT_C
    kv_ref[:, :C] = jnp.dot(xb, wk_ref[...], preferred_element_type=jnp.float32) + bk_ref[...]
    kv_ref[:, C:] = jnp.dot(xb, wv_ref[...], preferred_element_type=jnp.float32) + bv_ref[...]


def _combine_body(x_ref, ws_ref, bs_ref, num_ref, den_ref, o_ref):
    num = num_ref[0] + num_ref[1]
    den = jnp.sum(den_ref[...], axis=0)
    att = num / (den + 1e-16)
    skip = jnp.dot(x_ref[...], ws_ref[...], preferred_element_type=jnp.float32) + bs_ref[...]
    o_ref[...] = jnp.maximum(att + skip, 0.0)


def _edge_pass(q, kv, src, dst):
    mesh = plsc.VectorSubcoreMesh(core_axis_name="c", subcore_axis_name="s")

    @functools.partial(
        pl.kernel,
        mesh=mesh,
        out_type=[
            jax.ShapeDtypeStruct((NC, N_PAD, D), jnp.float32),
            jax.ShapeDtypeStruct((NW, N_PAD), jnp.float32),
        ],
        scratch_types=[
            pltpu.VMEM((E_SUB,), jnp.int32),
            pltpu.VMEM((E_SUB,), jnp.int32),
            pltpu.VMEM((E_SUB,), jnp.int32),
            pltpu.VMEM((E_SUB,), jnp.int32),
            pltpu.VMEM((E_SUB, D), jnp.float32),
            pltpu.VMEM((E_SUB, D), jnp.float32),
            pltpu.VMEM((E_SUB, 2 * D), jnp.float32),
            pltpu.VMEM((E_SUB, 2 * D), jnp.float32),
            pltpu.VMEM((E_SUB, D), jnp.float32),
            pltpu.VMEM((N_PAD,), jnp.float32),
            pltpu.VMEM_SHARED((N_PAD, D), jnp.float32),
            pltpu.SemaphoreType.DMA,
            pltpu.SemaphoreType.DMA,
            pltpu.SemaphoreType.DMA,
            pltpu.SemaphoreType.DMA,
        ],
    )
    def edge_kernel(q_hbm, kv_hbm, src_hbm, dst_hbm, num_hbm, den_hbm,
                    is0, is1, id0, id1, qb0, qb1, kvb0, kvb1,
                    sb, denw, num_sh, si0, si1, sg0, sg1):
        cid = lax.axis_index("c")
        sid = lax.axis_index("s")
        wid = sid * NC + cid
        lane = lax.iota(jnp.int32, 16)

        ibs = (is0, is1)
        ibd = (id0, id1)
        qbs = (qb0, qb1)
        kvbs = (kvb0, kvb1)
        sis = (si0, si1)
        sgs = (sg0, sg1)

        def zero_qb(i, _):
            for j in range(D // 16):
                qb0[i, pl.ds(j * 16, 16)] = jnp.zeros((16,), jnp.float32)
            return 0
        lax.fori_loop(0, E_SUB, zero_qb, 0)

        def zero_denw(i, _):
            denw[pl.ds(i * 16, 16)] = jnp.zeros((16,), jnp.float32)
            return 0
        lax.fori_loop(0, N_PAD // 16, zero_denw, 0)

        def zero_stripe(i, _):
            row0 = pl.multiple_of(sid * ROWS_PER_TILE + i * E_SUB, 8)
            pltpu.sync_copy(qb0, num_sh.at[pl.ds(row0, E_SUB)])
            return 0
        lax.fori_loop(0, ROWS_PER_TILE // E_SUB, zero_stripe, 0)

        plsc.subcore_barrier()

        def issue_idx(t, p):
            base = pl.multiple_of(wid * E_PER_W + t * E_SUB, 8)
            pltpu.async_copy(src_hbm.at[pl.ds(base, E_SUB)], ibs[p], sis[p])
            pltpu.async_copy(dst_hbm.at[pl.ds(base, E_SUB)], ibd[p], sis[p])

        def wait_idx(p):
            pltpu.make_async_copy(src_hbm.at[pl.ds(0, E_SUB)], ibs[p], sis[p]).wait()
            pltpu.make_async_copy(dst_hbm.at[pl.ds(0, E_SUB)], ibd[p], sis[p]).wait()

        def issue_gathers(p):
            pltpu.async_copy(q_hbm.at[ibd[p]], qbs[p], sgs[p])
            pltpu.async_copy(kv_hbm.at[ibs[p]], kvbs[p], sgs[p])

        def wait_gathers(p):
            pltpu.make_async_copy(q_hbm.at[pl.ds(0, E_SUB)], qbs[p], sgs[p]).wait()
            pltpu.make_async_copy(kv_hbm.at[pl.ds(0, E_SUB)], kvbs[p], sgs[p]).wait()

        perms = [lane ^ s for s in (8, 4, 2, 1)]
        gd = lax.GatherDimensionNumbers(
            offset_dims=(), collapsed_slice_dims=(0,), start_index_map=(0,))

        def lane_allsum(vv):
            for p in perms:
                vv = vv + lax.gather(
                    vv, p[:, None], gd, slice_sizes=(1,),
                    mode=lax.GatherScatterMode.PROMISE_IN_BOUNDS)
            return vv

        def compute(p):
            qb_, kvb_, ibd_ = qbs[p], kvbs[p], ibd[p]
            dvs = (ibd_[pl.ds(0, 16)], ibd_[pl.ds(16, 16)], ibd_[pl.ds(24, 16)])

            def emit_scale(rp, wp):
                vs = [kvb_[rp, pl.ds(C + j * 16, 16)] for j in range(D // 16)]
                for j in range(D // 16):
                    sb[rp, pl.ds(j * 16, 16)] = vs[j] * wp
                n = dvs[2][rp - 24] if rp >= 32 else dvs[rp // 16][rp % 16]
                o = pl.multiple_of((n >> 4) << 4, 16)
                plsc.addupdate(denw.at[pl.ds(o, 16)],
                               jnp.where(lane == (n & 15), wp, 0.0))

            prev = None
            for r in range(E_SUB):
                qs = [qb_[r, pl.ds(j * 16, 16)] for j in range(D // 16)]
                ks = [kvb_[r, pl.ds(j * 16, 16)] for j in range(D // 16)]
                acc = qs[0] * ks[0]
                for j in range(1, D // 16):
                    acc = acc + qs[j] * ks[j]
                if prev is not None:
                    emit_scale(*prev)
                w = jnp.exp(lane_allsum(acc))
                prev = (r, w)
            emit_scale(*prev)
            pltpu.sync_copy(sb, num_sh.at[ibd_], add=True)

        issue_idx(0, 0)
        wait_idx(0)
        issue_gathers(0)
        issue_idx(1, 1)

        def pair(m, _):
            t0 = m * 2
            not_last = m < N_PAIR - 1
            wait_idx(1)
            issue_gathers(1)
            wait_gathers(0)
            compute(0)

            @pl.when(not_last)
            def _():
                issue_idx(t0 + 2, 0)

            @pl.when(not_last)
            def _():
                wait_idx(0)
                issue_gathers(0)
            wait_gathers(1)
            compute(1)

            @pl.when(not_last)
            def _():
                issue_idx(t0 + 3, 1)
            return 0
        lax.fori_loop(0, N_PAIR, pair, 0)

        pltpu.sync_copy(denw, den_hbm.at[wid])
        plsc.subcore_barrier()

        def dump_stripe(t, _):
            row0 = pl.multiple_of(sid * ROWS_PER_TILE + t * ZROWS, 8)
            pltpu.sync_copy(num_sh.at[pl.ds(row0, ZROWS)],
                            num_hbm.at[cid, pl.ds(row0, ZROWS)])
            return 0
        lax.fori_loop(0, ROWS_PER_TILE // ZROWS, dump_stripe, 0)

    return edge_kernel(q, kv, src, dst)


def kernel(x, edge_index, Wq, bq, Wk, bk, Wv, bv, Ws, bs):
    src = edge_index[0].astype(jnp.int32)
    dst = edge_index[1].astype(jnp.int32)
    bq2 = bq.reshape(1, C).astype(jnp.float32)
    bk2 = bk.reshape(1, C).astype(jnp.float32)
    bv2 = bv.reshape(1, C).astype(jnp.float32)
    bs2 = bs.reshape(1, C).astype(jnp.float32)

    grid = N_NODES // BLK
    row_spec = pl.BlockSpec((BLK, D), lambda i: (i, 0))
    w_spec = pl.BlockSpec((D, C), lambda i: (0, 0))
    b_spec = pl.BlockSpec((1, C), lambda i: (0, 0))

    q, kv = pl.pallas_call(
        _proj_body,
        grid=(grid,),
        in_specs=[row_spec, w_spec, b_spec, w_spec, b_spec, w_spec, b_spec],
        out_specs=[row_spec, pl.BlockSpec((BLK, 2 * C), lambda i: (i, 0))],
        out_shape=[jax.ShapeDtypeStruct((N_NODES, C), jnp.float32),
                   jax.ShapeDtypeStruct((N_NODES, 2 * C), jnp.float32)],
    )(x, Wq, bq2, Wk, bk2, Wv, bv2)

    num, den = _edge_pass(q, kv, src, dst)
    den_col = den.reshape(NW, N_PAD, 1)

    out = pl.pallas_call(
        _combine_body,
        grid=(grid,),
        in_specs=[row_spec, w_spec, b_spec,
                  pl.BlockSpec((NC, BLK, D), lambda i: (0, i, 0)),
                  pl.BlockSpec((NW, BLK, 1), lambda i: (0, i, 0))],
        out_specs=pl.BlockSpec((BLK, C), lambda i: (i, 0)),
        out_shape=jax.ShapeDtypeStruct((N_NODES, C), jnp.float32),
    )(x, Ws, bs2, num, den_col)
    return out


# speedup vs baseline: 1.3796x; 1.3796x over previous
import functools

import jax
import jax.numpy as jnp
from jax import lax
from jax.experimental import pallas as pl
from jax.experimental.pallas import tpu as pltpu
from jax.experimental.pallas import tpu_sc as plsc

N_NODES = 10000
N_EDGES = 320000
D = 128
C = 128
INV_SQRT_C = 1.0 / (C ** 0.5)

NC = 2
NS = 16
NW = NC * NS
E_PER_W = N_EDGES // NW
E_SUB = 40
N_SUB = E_PER_W // E_SUB
N_PAIR = N_SUB // 2
N_PAD = 10240
ROWS_PER_TILE = N_PAD // NS
ZROWS = 128

BLK = 400


def _proj_body(x_ref, wq_ref, bq_ref, wk_ref, bk_ref, wv_ref, bv_ref,
               q_ref, kv_ref):
    xb = x_ref[...]
    q = jnp.dot(xb, wq_ref[...], preferred_element_type=jnp.float32) + bq_ref[...]
    q_ref[...] = q * INV_SQRT_C
    kv_ref[:, :C] = jnp.dot(xb, wk_ref[...], preferred_element_type=jnp.float32) + bk_ref[...]
    kv_ref[:, C:] = jnp.dot(xb, wv_ref[...], preferred_element_type=jnp.float32) + bv_ref[...]


def _combine_body(x_ref, ws_ref, bs_ref, num_ref, den_ref, o_ref):
    num = num_ref[0] + num_ref[1]
    den = jnp.sum(den_ref[...], axis=0)
    att = num / (den + 1e-16)
    skip = jnp.dot(x_ref[...], ws_ref[...], preferred_element_type=jnp.float32) + bs_ref[...]
    o_ref[...] = jnp.maximum(att + skip, 0.0)


def _edge_pass(q, kv, src, dst):
    mesh = plsc.VectorSubcoreMesh(core_axis_name="c", subcore_axis_name="s")

    @functools.partial(
        pl.kernel,
        mesh=mesh,
        out_type=[
            jax.ShapeDtypeStruct((NC, N_PAD, D), jnp.float32),
            jax.ShapeDtypeStruct((NW, N_PAD), jnp.float32),
        ],
        scratch_types=[
            pltpu.VMEM((E_SUB,), jnp.int32),
            pltpu.VMEM((E_SUB,), jnp.int32),
            pltpu.VMEM((E_SUB,), jnp.int32),
            pltpu.VMEM((E_SUB,), jnp.int32),
            pltpu.VMEM((E_SUB, D), jnp.float32),
            pltpu.VMEM((E_SUB, D), jnp.float32),
            pltpu.VMEM((E_SUB, 2 * D), jnp.float32),
            pltpu.VMEM((E_SUB, 2 * D), jnp.float32),
            pltpu.VMEM((E_SUB, D), jnp.float32),
            pltpu.VMEM((E_SUB,), jnp.int32),
            pltpu.VMEM((N_PAD,), jnp.float32),
            pltpu.VMEM_SHARED((N_PAD, D), jnp.float32),
            pltpu.SemaphoreType.DMA,
            pltpu.SemaphoreType.DMA,
            pltpu.SemaphoreType.DMA,
            pltpu.SemaphoreType.DMA,
            pltpu.SemaphoreType.DMA,
        ],
    )
    def edge_kernel(q_hbm, kv_hbm, src_hbm, dst_hbm, num_hbm, den_hbm,
                    is0, is1, id0, id1, qb0, qb1, kvb0, kvb1,
                    sb, sbi, denw, num_sh, si0, si1, sg0, sg1, ss):
        cid = lax.axis_index("c")
        sid = lax.axis_index("s")
        wid = sid * NC + cid
        lane = lax.iota(jnp.int32, 16)

        ibs = (is0, is1)
        ibd = (id0, id1)
        qbs = (qb0, qb1)
        kvbs = (kvb0, kvb1)
        sis = (si0, si1)
        sgs = (sg0, sg1)

        def zero_qb(i, _):
            for j in range(D // 16):
                qb0[i, pl.ds(j * 16, 16)] = jnp.zeros((16,), jnp.float32)
            return 0
        lax.fori_loop(0, E_SUB, zero_qb, 0)

        def zero_denw(i, _):
            denw[pl.ds(i * 16, 16)] = jnp.zeros((16,), jnp.float32)
            return 0
        lax.fori_loop(0, N_PAD // 16, zero_denw, 0)

        def zero_stripe(i, _):
            row0 = pl.multiple_of(sid * ROWS_PER_TILE + i * E_SUB, 8)
            pltpu.sync_copy(qb0, num_sh.at[pl.ds(row0, E_SUB)])
            return 0
        lax.fori_loop(0, ROWS_PER_TILE // E_SUB, zero_stripe, 0)

        plsc.subcore_barrier()

        def issue_idx(t, p):
            base = pl.multiple_of(wid * E_PER_W + t * E_SUB, 8)
            pltpu.async_copy(src_hbm.at[pl.ds(base, E_SUB)], ibs[p], sis[p])
            pltpu.async_copy(dst_hbm.at[pl.ds(base, E_SUB)], ibd[p], sis[p])

        def wait_idx(p):
            pltpu.make_async_copy(src_hbm.at[pl.ds(0, E_SUB)], ibs[p], sis[p]).wait()
            pltpu.make_async_copy(dst_hbm.at[pl.ds(0, E_SUB)], ibd[p], sis[p]).wait()

        def issue_gathers(p):
            pltpu.async_copy(q_hbm.at[ibd[p]], qbs[p], sgs[p])
            pltpu.async_copy(kv_hbm.at[ibs[p]], kvbs[p], sgs[p])

        def wait_gathers(p):
            pltpu.make_async_copy(q_hbm.at[pl.ds(0, E_SUB)], qbs[p], sgs[p]).wait()
            pltpu.make_async_copy(kv_hbm.at[pl.ds(0, E_SUB)], kvbs[p], sgs[p]).wait()

        perms = [lane ^ s for s in (8, 4, 2, 1)]
        gd = lax.GatherDimensionNumbers(
            offset_dims=(), collapsed_slice_dims=(0,), start_index_map=(0,))

        def lane_allsum(vv):
            for p in perms:
                vv = vv + lax.gather(
                    vv, p[:, None], gd, slice_sizes=(1,),
                    mode=lax.GatherScatterMode.PROMISE_IN_BOUNDS)
            return vv

        def do_edges(qb_, kvb_, dvec, ebase, li0, ne):
            def emit_scale(rp, wp, ip):
                vs = [kvb_[rp, pl.ds(C + j * 16, 16)] for j in range(D // 16)]
                for j in range(D // 16):
                    sb[rp, pl.ds(j * 16, 16)] = vs[j] * wp
                n = dvec[li0 + ip]
                o = pl.multiple_of((n >> 4) << 4, 16)
                plsc.addupdate(denw.at[pl.ds(o, 16)],
                               jnp.where(lane == (n & 15), wp, 0.0))

            prev = None
            for i2 in range(ne):
                r = ebase + i2
                qs = [qb_[r, pl.ds(j * 16, 16)] for j in range(D // 16)]
                ks = [kvb_[r, pl.ds(j * 16, 16)] for j in range(D // 16)]
                acc = qs[0] * ks[0]
                for j in range(1, D // 16):
                    acc = acc + qs[j] * ks[j]
                if prev is not None:
                    emit_scale(*prev)
                w = jnp.exp(lane_allsum(acc))
                prev = (r, w, i2)
            emit_scale(*prev)

        def wait_scatter():
            pltpu.make_async_copy(q_hbm.at[pl.ds(0, E_SUB)], sb, ss).wait()

        def compute(p, wait_prev):
            qb_, kvb_, ibd_ = qbs[p], kvbs[p], ibd[p]

            @pl.when(wait_prev)
            def _():
                wait_scatter()
            sbi[pl.ds(0, 16)] = ibd_[pl.ds(0, 16)]
            sbi[pl.ds(16, 16)] = ibd_[pl.ds(16, 16)]
            sbi[pl.ds(24, 16)] = ibd_[pl.ds(24, 16)]

            def group16(g, _):
                dvec = ibd_[pl.ds(g * 16, 16)]
                do_edges(qb_, kvb_, dvec, g * 16, 0, 16)
                return 0
            lax.fori_loop(0, 2, group16, 0)
            dvec = ibd_[pl.ds(24, 16)]
            do_edges(qb_, kvb_, dvec, 32, 8, 8)
            pltpu.async_copy(sb, num_sh.at[sbi], ss, add=True)

        issue_idx(0, 0)
        wait_idx(0)
        issue_gathers(0)
        issue_idx(1, 1)

        def pair(m, _):
            t0 = m * 2
            not_last = m < N_PAIR - 1
            wait_idx(1)
            issue_gathers(1)
            wait_gathers(0)
            compute(0, t0 >= 1)

            @pl.when(not_last)
            def _():
                issue_idx(t0 + 2, 0)

            @pl.when(not_last)
            def _():
                wait_idx(0)
                issue_gathers(0)
            wait_gathers(1)
            compute(1, t0 >= 0)

            @pl.when(not_last)
            def _():
                issue_idx(t0 + 3, 1)
            return 0
        lax.fori_loop(0, N_PAIR, pair, 0)

        wait_scatter()
        pltpu.sync_copy(denw, den_hbm.at[wid])
        plsc.subcore_barrier()

        def dump_stripe(t, _):
            row0 = pl.multiple_of(sid * ROWS_PER_TILE + t * ZROWS, 8)
            pltpu.sync_copy(num_sh.at[pl.ds(row0, ZROWS)],
                            num_hbm.at[cid, pl.ds(row0, ZROWS)])
            return 0
        lax.fori_loop(0, ROWS_PER_TILE // ZROWS, dump_stripe, 0)

    return edge_kernel(q, kv, src, dst)


def kernel(x, edge_index, Wq, bq, Wk, bk, Wv, bv, Ws, bs):
    src = edge_index[0].astype(jnp.int32)
    dst = edge_index[1].astype(jnp.int32)
    bq2 = bq.reshape(1, C).astype(jnp.float32)
    bk2 = bk.reshape(1, C).astype(jnp.float32)
    bv2 = bv.reshape(1, C).astype(jnp.float32)
    bs2 = bs.reshape(1, C).astype(jnp.float32)

    grid = N_NODES // BLK
    row_spec = pl.BlockSpec((BLK, D), lambda i: (i, 0))
    w_spec = pl.BlockSpec((D, C), lambda i: (0, 0))
    b_spec = pl.BlockSpec((1, C), lambda i: (0, 0))

    q, kv = pl.pallas_call(
        _proj_body,
        grid=(grid,),
        in_specs=[row_spec, w_spec, b_spec, w_spec, b_spec, w_spec, b_spec],
        out_specs=[row_spec, pl.BlockSpec((BLK, 2 * C), lambda i: (i, 0))],
        out_shape=[jax.ShapeDtypeStruct((N_NODES, C), jnp.float32),
                   jax.ShapeDtypeStruct((N_NODES, 2 * C), jnp.float32)],
    )(x, Wq, bq2, Wk, bk2, Wv, bv2)

    num, den = _edge_pass(q, kv, src, dst)
    den_col = den.reshape(NW, N_PAD, 1)

    out = pl.pallas_call(
        _combine_body,
        grid=(grid,),
        in_specs=[row_spec, w_spec, b_spec,
                  pl.BlockSpec((NC, BLK, D), lambda i: (0, i, 0)),
                  pl.BlockSpec((NW, BLK, 1), lambda i: (0, i, 0))],
        out_specs=pl.BlockSpec((BLK, C), lambda i: (i, 0)),
        out_shape=jax.ShapeDtypeStruct((N_NODES, C), jnp.float32),
    )(x, Ws, bs2, num, den_col)
    return out
